# Initial kernel scaffold; baseline (speedup 1.0000x reference)
import jax
import jax.numpy as jnp
from jax.experimental import pallas as pl


def kernel(cur_proba, proba, is_ended, state, decodeds):
    raise NotImplementedError("write your pallas kernel here")


# speedup vs baseline: 2.2849x; 2.2849x over previous
import math

import jax
import jax.numpy as jnp
from jax import lax
from jax.experimental import pallas as pl

_END = 2


def _body(cur_ref, pcol_ref, ecol_ref, erow_ref, state_ref, dec_ref,
          outp_ref, outv_ref, oute_ref, outd_ref, outs_ref):
    beam, V = cur_ref.shape[1], cur_ref.shape[2]
    t = dec_ref.shape[1]
    x = cur_ref[0]
    pcol = pcol_ref[0]
    ecol = ecol_ref[0]
    erow = erow_ref[0]
    st = state_ref[0]
    dec = dec_ref[0].astype(jnp.float32)

    neg = jnp.float32(-jnp.inf)
    s = jnp.sum(jnp.exp(x), axis=-1, keepdims=True)
    c = pcol - jnp.log(s)

    col = lax.broadcasted_iota(jnp.int32, (beam, V), 1)
    bsub = lax.broadcasted_iota(jnp.int32, (beam, V), 0)
    flat = bsub * V + col

    total = x + c
    is_end = ecol > 0
    total = jnp.where(is_end, jnp.where(col == _END, pcol, neg), total)

    l8 = lax.broadcasted_iota(jnp.int32, (1, beam), 1)
    sub8 = lax.broadcasted_iota(jnp.int32, (beam, beam), 0)
    lan8 = lax.broadcasted_iota(jnp.int32, (beam, beam), 1)
    val_row = jnp.zeros((1, beam), jnp.float32)
    voc_row = jnp.zeros((1, beam), jnp.int32)
    W = jnp.zeros((beam, beam), jnp.float32)
    BIG = jnp.int32(1 << 30)
    vlog2 = int(math.log2(V))

    for k in range(beam):
        m = jnp.max(total)
        fi = jnp.min(jnp.where(total == m, flat, BIG))
        total = jnp.where(flat == fi, neg, total)
        vk = fi & (V - 1) if (1 << vlog2) == V else fi % V
        bk = lax.shift_right_logical(fi, vlog2) if (1 << vlog2) == V else fi // V
        val_row = jnp.where(l8 == k, m, val_row)
        voc_row = jnp.where(l8 == k, vk, voc_row)
        W = W + jnp.where((sub8 == k) & (lan8 == bk), 1.0, 0.0)

    outp_ref[0] = val_row
    outv_ref[0] = voc_row
    oute_ref[0] = ((erow > 0) | (voc_row == _END)).astype(jnp.int32)
    gathered = lax.dot_general(dec, W, (((1,), (1,)), ((), ())),
                               preferred_element_type=jnp.float32)
    outd_ref[0] = gathered.astype(jnp.int32)
    outs_ref[0] = lax.dot_general(W, st, (((1,), (0,)), ((), ())),
                                  preferred_element_type=jnp.float32)


def kernel(cur_proba, proba, is_ended, state, decodeds):
    batch, beam, V = cur_proba.shape
    d = state.shape[-1]
    t = decodeds.shape[0]
    pcol = proba.reshape(batch, beam, 1)
    ecol = is_ended.astype(jnp.int32).reshape(batch, beam, 1)
    erow = is_ended.astype(jnp.int32).reshape(batch, 1, beam)
    dec3 = decodeds.astype(jnp.int32).transpose(1, 0, 2)

    grid = (batch,)
    outs = pl.pallas_call(
        _body,
        grid=grid,
        in_specs=[
            pl.BlockSpec((1, beam, V), lambda b: (b, 0, 0)),
            pl.BlockSpec((1, beam, 1), lambda b: (b, 0, 0)),
            pl.BlockSpec((1, beam, 1), lambda b: (b, 0, 0)),
            pl.BlockSpec((1, 1, beam), lambda b: (b, 0, 0)),
            pl.BlockSpec((1, beam, d), lambda b: (b, 0, 0)),
            pl.BlockSpec((1, t, beam), lambda b: (b, 0, 0)),
        ],
        out_specs=[
            pl.BlockSpec((1, 1, beam), lambda b: (b, 0, 0)),
            pl.BlockSpec((1, 1, beam), lambda b: (b, 0, 0)),
            pl.BlockSpec((1, 1, beam), lambda b: (b, 0, 0)),
            pl.BlockSpec((1, t, beam), lambda b: (b, 0, 0)),
            pl.BlockSpec((1, beam, d), lambda b: (b, 0, 0)),
        ],
        out_shape=[
            jax.ShapeDtypeStruct((batch, 1, beam), jnp.float32),
            jax.ShapeDtypeStruct((batch, 1, beam), jnp.int32),
            jax.ShapeDtypeStruct((batch, 1, beam), jnp.int32),
            jax.ShapeDtypeStruct((batch, t, beam), jnp.int32),
            jax.ShapeDtypeStruct((batch, beam, d), jnp.float32),
        ],
    )(cur_proba, pcol, ecol, erow, state, dec3)

    new_proba3, voc3, ended3, decg3, new_state = outs
    new_proba = new_proba3.reshape(batch, beam)
    topk_voc = voc3.reshape(batch, beam)
    new_is_ended = ended3.reshape(batch, beam).astype(bool)
    gathered_dec = decg3.transpose(1, 0, 2)
    new_decodeds = jnp.concatenate([gathered_dec, topk_voc[None]], axis=0)
    cur_input = topk_voc.reshape(-1)
    return new_proba, new_decodeds, new_is_ended, new_state, cur_input


# speedup vs baseline: 3.6757x; 1.6087x over previous
import math

import jax
import jax.numpy as jnp
from jax import lax
from jax.experimental import pallas as pl
from jax.experimental.pallas import tpu as pltpu

_END = 2
_LANES = 128


def _body(cur_ref, pcol_ref, ecol_ref, erow_ref, state_ref, dec_ref,
          outp_ref, outv_ref, oute_ref, outd_ref, outs_ref,
          m_s, fi_s):
    beam, V = cur_ref.shape[1], cur_ref.shape[2]
    nchunk = V // _LANES
    x = cur_ref[0]
    pcol = pcol_ref[0]
    ecol = ecol_ref[0]
    erow = erow_ref[0]
    st = state_ref[0]
    dec = dec_ref[0].astype(jnp.float32)

    neg = jnp.float32(-jnp.inf)
    BIG = jnp.int32(1 << 30)

    s = jnp.sum(jnp.exp(x), axis=-1, keepdims=True)
    c = pcol - jnp.log(s)

    x3 = x.reshape(beam, nchunk, _LANES)
    ch = lax.broadcasted_iota(jnp.int32, (beam, nchunk, _LANES), 1)

    m1 = jnp.max(x3, axis=1)
    a1 = jnp.min(jnp.where(x3 == m1[:, None, :], ch, BIG), axis=1)
    x3m = jnp.where(ch == a1[:, None, :], neg, x3)
    m2 = jnp.max(x3m, axis=1)
    a2 = jnp.min(jnp.where(x3m == m2[:, None, :], ch, BIG), axis=1)

    bsub = lax.broadcasted_iota(jnp.int32, (beam, _LANES), 0)
    lane = lax.broadcasted_iota(jnp.int32, (beam, _LANES), 1)
    base = bsub * V + lane
    h1 = m1 + c
    f1 = base + a1 * _LANES
    h2 = m2 + c
    f2 = base + a2 * _LANES

    endm = ecol > 0
    e_lane = lane == (_END % _LANES)
    e_flat = bsub * V + _END
    h1 = jnp.where(endm, jnp.where(e_lane, pcol, neg), h1)
    f1 = jnp.where(endm, jnp.where(e_lane, e_flat, BIG), f1)
    h2 = jnp.where(endm, neg, h2)
    f2 = jnp.where(endm, BIG, f2)

    t_cnt = jnp.zeros((beam, _LANES), jnp.int32)
    danger = neg
    deg = jnp.bool_(False)

    l8 = lax.broadcasted_iota(jnp.int32, (1, beam), 1)
    sub8 = lax.broadcasted_iota(jnp.int32, (beam, beam), 0)
    lan8 = lax.broadcasted_iota(jnp.int32, (beam, beam), 1)
    val_row = jnp.zeros((1, beam), jnp.float32)
    voc_row = jnp.zeros((1, beam), jnp.int32)
    W = jnp.zeros((beam, beam), jnp.float32)
    vlog2 = int(math.log2(V))
    removed = []

    for k in range(beam):
        m_fast = jnp.max(h1)
        safe = jnp.logical_and(jnp.logical_not(deg), m_fast > danger)

        @pl.when(safe)
        def _(h1=h1, f1=f1, m_fast=m_fast):
            fi_f = jnp.min(jnp.where(h1 == m_fast, f1, BIG))
            m_s[0] = m_fast
            fi_s[0] = fi_f

        @pl.when(jnp.logical_not(safe))
        def _(removed=tuple(removed)):
            f3 = lax.broadcasted_iota(jnp.int32, (beam, nchunk, _LANES), 0) * V \
                + lax.broadcasted_iota(jnp.int32, (beam, nchunk, _LANES), 1) * _LANES \
                + lax.broadcasted_iota(jnp.int32, (beam, nchunk, _LANES), 2)
            t3 = x3 + c[:, None]
            e3 = endm[:, :, None]
            bflat = lax.broadcasted_iota(jnp.int32, (beam, 1, 1), 0) * V + _END
            t3 = jnp.where(e3, jnp.where(f3 == bflat, pcol[:, None], neg), t3)
            rm = jnp.zeros((beam, nchunk, _LANES), jnp.bool_)
            for r in removed:
                rm = jnp.logical_or(rm, f3 == r)
            t3 = jnp.where(rm, neg, t3)
            m_slow = jnp.max(t3)
            fi_slow = jnp.min(jnp.where(t3 == m_slow, f3, BIG))
            m_s[0] = m_slow
            fi_s[0] = fi_slow

        m = m_s[0]
        fi = fi_s[0]
        removed.append(fi)

        colm = f1 == fi
        second_pop = jnp.any(jnp.logical_and(colm, t_cnt == 1))
        danger = jnp.where(jnp.logical_and(safe, second_pop),
                           jnp.maximum(danger, m), danger)
        t_cnt = t_cnt + colm.astype(jnp.int32)
        h1 = jnp.where(colm, h2, h1)
        f1 = jnp.where(colm, f2, f1)
        h2 = jnp.where(colm, neg, h2)
        f2 = jnp.where(colm, BIG, f2)
        deg = jnp.logical_or(deg, jnp.logical_not(safe))

        vk = fi & (V - 1) if (1 << vlog2) == V else fi % V
        bk = lax.shift_right_logical(fi, vlog2) if (1 << vlog2) == V else fi // V
        val_row = jnp.where(l8 == k, m, val_row)
        voc_row = jnp.where(l8 == k, vk, voc_row)
        W = W + jnp.where((sub8 == k) & (lan8 == bk), 1.0, 0.0)

    outp_ref[0] = val_row
    outv_ref[0] = voc_row
    oute_ref[0] = ((erow > 0) | (voc_row == _END)).astype(jnp.int32)
    gathered = lax.dot_general(dec, W, (((1,), (1,)), ((), ())),
                               preferred_element_type=jnp.float32)
    outd_ref[0] = gathered.astype(jnp.int32)
    outs_ref[0] = lax.dot_general(W, st, (((1,), (0,)), ((), ())),
                                  preferred_element_type=jnp.float32)


def kernel(cur_proba, proba, is_ended, state, decodeds):
    batch, beam, V = cur_proba.shape
    d = state.shape[-1]
    t = decodeds.shape[0]
    pcol = proba.reshape(batch, beam, 1)
    ecol = is_ended.astype(jnp.int32).reshape(batch, beam, 1)
    erow = is_ended.astype(jnp.int32).reshape(batch, 1, beam)
    dec3 = decodeds.astype(jnp.int32).transpose(1, 0, 2)

    outs = pl.pallas_call(
        _body,
        grid=(batch,),
        in_specs=[
            pl.BlockSpec((1, beam, V), lambda b: (b, 0, 0)),
            pl.BlockSpec((1, beam, 1), lambda b: (b, 0, 0)),
            pl.BlockSpec((1, beam, 1), lambda b: (b, 0, 0)),
            pl.BlockSpec((1, 1, beam), lambda b: (b, 0, 0)),
            pl.BlockSpec((1, beam, d), lambda b: (b, 0, 0)),
            pl.BlockSpec((1, t, beam), lambda b: (b, 0, 0)),
        ],
        out_specs=[
            pl.BlockSpec((1, 1, beam), lambda b: (b, 0, 0)),
            pl.BlockSpec((1, 1, beam), lambda b: (b, 0, 0)),
            pl.BlockSpec((1, 1, beam), lambda b: (b, 0, 0)),
            pl.BlockSpec((1, t, beam), lambda b: (b, 0, 0)),
            pl.BlockSpec((1, beam, d), lambda b: (b, 0, 0)),
        ],
        out_shape=[
            jax.ShapeDtypeStruct((batch, 1, beam), jnp.float32),
            jax.ShapeDtypeStruct((batch, 1, beam), jnp.int32),
            jax.ShapeDtypeStruct((batch, 1, beam), jnp.int32),
            jax.ShapeDtypeStruct((batch, t, beam), jnp.int32),
            jax.ShapeDtypeStruct((batch, beam, d), jnp.float32),
        ],
        scratch_shapes=[
            pltpu.SMEM((1,), jnp.float32),
            pltpu.SMEM((1,), jnp.int32),
        ],
    )(cur_proba, pcol, ecol, erow, state, dec3)

    new_proba3, voc3, ended3, decg3, new_state = outs
    new_proba = new_proba3.reshape(batch, beam)
    topk_voc = voc3.reshape(batch, beam)
    new_is_ended = ended3.reshape(batch, beam).astype(bool)
    gathered_dec = decg3.transpose(1, 0, 2)
    new_decodeds = jnp.concatenate([gathered_dec, topk_voc[None]], axis=0)
    cur_input = topk_voc.reshape(-1)
    return new_proba, new_decodeds, new_is_ended, new_state, cur_input
